# Initial kernel scaffold; baseline (speedup 1.0000x reference)
import jax
import jax.numpy as jnp
from jax.experimental import pallas as pl


def kernel(z, edge_index):
    raise NotImplementedError("write your pallas kernel here")


# speedup vs baseline: 3.4305x; 3.4305x over previous
import functools

import jax
import jax.numpy as jnp
from jax import lax
from jax.experimental import pallas as pl
from jax.experimental.pallas import tpu as pltpu
from jax.experimental.pallas import tpu_sc as plsc

N_NODES = 10000
DIM = 128
N_EDGES = 320000
LANES = 16

_info = plsc.get_sparse_core_info()
NC = _info.num_cores
NS = _info.num_subcores
NW = NC * NS

EDGES_PER_W = N_EDGES // NW
CHUNK = 80
ROUNDS = EDGES_PER_W // CHUNK
GROUPS = CHUNK // LANES


def _body(z_hbm, src_hbm, dst_hbm, out_hbm,
          sidx, didx, srows, drows, ps, outv, sem_s, sem_d):
    wid = lax.axis_index("s") * NC + lax.axis_index("c")
    base_w = wid * EDGES_PER_W

    lanes_i = lax.iota(jnp.int32, LANES)

    def round_body(r, carry):
        base = pl.multiple_of(base_w + r * CHUNK, 8)
        pltpu.sync_copy(src_hbm.at[pl.ds(base, CHUNK)], sidx)
        pltpu.sync_copy(dst_hbm.at[pl.ds(base, CHUNK)], didx)
        cp_s = pltpu.async_copy(z_hbm.at[sidx], srows, sem_s)
        cp_d = pltpu.async_copy(z_hbm.at[didx], drows, sem_d)
        cp_s.wait()
        cp_d.wait()

        def group_body(g, carry2):
            e0 = g * LANES
            for e in range(LANES):
                p = (srows[e0 + e, pl.ds(0, LANES)]
                     * drows[e0 + e, pl.ds(0, LANES)])
                for j in range(1, DIM // LANES):
                    p += (srows[e0 + e, pl.ds(j * LANES, LANES)]
                          * drows[e0 + e, pl.ds(j * LANES, LANES)])
                ps[pl.ds(e * LANES, LANES)] = p
            flat = lanes_i * LANES
            acc = plsc.load_gather(ps, [flat])
            for j in range(1, LANES):
                acc += plsc.load_gather(ps, [flat + j])
            outv[pl.ds(e0, LANES)] = 1.0 / (1.0 + jnp.exp(-acc))
            return carry2

        lax.fori_loop(0, GROUPS, group_body, 0, unroll=False)
        pltpu.sync_copy(outv, out_hbm.at[pl.ds(base, CHUNK)])
        return carry

    lax.fori_loop(0, ROUNDS, round_body, 0, unroll=False)


@jax.jit
def _decode(z, src, dst):
    mesh = plsc.VectorSubcoreMesh(core_axis_name="c", subcore_axis_name="s")
    return pl.kernel(
        _body,
        out_type=jax.ShapeDtypeStruct((N_EDGES,), jnp.float32),
        mesh=mesh,
        compiler_params=pltpu.CompilerParams(needs_layout_passes=False),
        scratch_types=[
            pltpu.VMEM((CHUNK,), jnp.int32),
            pltpu.VMEM((CHUNK,), jnp.int32),
            pltpu.VMEM((CHUNK, DIM), jnp.float32),
            pltpu.VMEM((CHUNK, DIM), jnp.float32),
            pltpu.VMEM((LANES * LANES,), jnp.float32),
            pltpu.VMEM((CHUNK,), jnp.float32),
            pltpu.SemaphoreType.DMA,
            pltpu.SemaphoreType.DMA,
        ],
    )(z, src, dst)


def kernel(z, edge_index):
    src = edge_index[0]
    dst = edge_index[1]
    return _decode(z, src, dst)


# speedup vs baseline: 7.5583x; 2.2033x over previous
import jax
import jax.numpy as jnp
from jax import lax
from jax.experimental import pallas as pl
from jax.experimental.pallas import tpu as pltpu
from jax.experimental.pallas import tpu_sc as plsc

N_NODES = 10000
DIM = 128
N_EDGES = 320000
LANES = 16

_info = plsc.get_sparse_core_info()
NC = _info.num_cores
NS = _info.num_subcores
NW = NC * NS

EDGES_PER_W = N_EDGES // NW
CHUNK = 80
ROUNDS = EDGES_PER_W // CHUNK
GROUPS = CHUNK // LANES


def _body(z_hbm, src_hbm, dst_hbm, out_hbm,
          sidx, didx, srows_a, drows_a, srows_b, drows_b, ps, outv,
          sem_sa, sem_da, sem_sb, sem_db):
    wid = lax.axis_index("s") * NC + lax.axis_index("c")
    base_w = wid * EDGES_PER_W

    lanes_i = lax.iota(jnp.int32, LANES)

    pltpu.sync_copy(src_hbm.at[wid], sidx)
    pltpu.sync_copy(dst_hbm.at[wid], didx)

    def fire(r, srows, drows, sem_s, sem_d):
        pltpu.async_copy(z_hbm.at[sidx.at[r]], srows, sem_s)
        pltpu.async_copy(z_hbm.at[didx.at[r]], drows, sem_d)

    def drain(srows, drows, sem_s, sem_d):
        pltpu.make_async_copy(z_hbm.at[sidx.at[0]], srows, sem_s).wait()
        pltpu.make_async_copy(z_hbm.at[didx.at[0]], drows, sem_d).wait()

    def compute(r, srows, drows):
        def group_body(g, carry):
            e0 = g * LANES
            for e in range(LANES):
                p = (srows[e0 + e, pl.ds(0, LANES)]
                     * drows[e0 + e, pl.ds(0, LANES)])
                for j in range(1, DIM // LANES):
                    p += (srows[e0 + e, pl.ds(j * LANES, LANES)]
                          * drows[e0 + e, pl.ds(j * LANES, LANES)])
                ps[pl.ds(e * LANES, LANES)] = p
            flat = lanes_i * LANES
            acc = plsc.load_gather(ps, [flat])
            for j in range(1, LANES):
                acc += plsc.load_gather(ps, [flat + j])
            outv[pl.ds(r * CHUNK + e0, LANES)] = 1.0 / (1.0 + jnp.exp(-acc))
            return carry

        lax.fori_loop(0, GROUPS, group_body, 0, unroll=False)

    fire(0, srows_a, drows_a, sem_sa, sem_da)

    def pair_body(i, carry):
        r0 = 2 * i
        fire(r0 + 1, srows_b, drows_b, sem_sb, sem_db)
        drain(srows_a, drows_a, sem_sa, sem_da)
        compute(r0, srows_a, drows_a)
        fire(r0 + 2, srows_a, drows_a, sem_sa, sem_da)
        drain(srows_b, drows_b, sem_sb, sem_db)
        compute(r0 + 1, srows_b, drows_b)
        return carry

    lax.fori_loop(0, (ROUNDS - 1) // 2, pair_body, 0, unroll=False)
    drain(srows_a, drows_a, sem_sa, sem_da)
    compute(ROUNDS - 1, srows_a, drows_a)

    pltpu.sync_copy(outv, out_hbm.at[pl.ds(base_w, EDGES_PER_W)])


@jax.jit
def _decode(z, src, dst):
    mesh = plsc.VectorSubcoreMesh(core_axis_name="c", subcore_axis_name="s")
    return pl.kernel(
        _body,
        out_type=jax.ShapeDtypeStruct((N_EDGES,), jnp.float32),
        mesh=mesh,
        compiler_params=pltpu.CompilerParams(needs_layout_passes=False),
        scratch_types=[
            pltpu.VMEM((ROUNDS, CHUNK), jnp.int32),
            pltpu.VMEM((ROUNDS, CHUNK), jnp.int32),
            pltpu.VMEM((CHUNK, DIM), jnp.float32),
            pltpu.VMEM((CHUNK, DIM), jnp.float32),
            pltpu.VMEM((CHUNK, DIM), jnp.float32),
            pltpu.VMEM((CHUNK, DIM), jnp.float32),
            pltpu.VMEM((LANES * LANES,), jnp.float32),
            pltpu.VMEM((EDGES_PER_W,), jnp.float32),
            pltpu.SemaphoreType.DMA,
            pltpu.SemaphoreType.DMA,
            pltpu.SemaphoreType.DMA,
            pltpu.SemaphoreType.DMA,
        ],
    )(z, src, dst)


def kernel(z, edge_index):
    src = edge_index[0].reshape(NW, ROUNDS, CHUNK)
    dst = edge_index[1].reshape(NW, ROUNDS, CHUNK)
    return _decode(z, src, dst)
